# baseline (speedup 1.0000x reference)
import functools

import jax
import jax.numpy as jnp
from jax import lax
from jax.experimental import pallas as pl
from jax.experimental.pallas import tpu as pltpu
from jax.experimental.pallas import tpu_sc as plsc

N_POS = 1024
SUB = 128
SIZE = 768
NSEG = 6
LANES = 16


def _sc_workers():
    try:
        info = plsc.get_sparse_core_info()
        return info.num_cores, info.num_subcores
    except Exception:
        return 2, 16


def kernel(xmin, ymin, xmax, ymax, width, height, first_page, last_page,
           x_table, y_table, w_table, h_table,
           first_page_embedding, last_page_embedding):
    B, L = xmin.shape
    NB = B * L
    NC, NS = _sc_workers()
    NW = NC * NS
    rows_per_w = NB // NW
    R = 128
    n_chunks = rows_per_w // R
    KPC = R * NSEG
    KW = rows_per_w * NSEG
    assert rows_per_w * NW == NB and n_chunks * R == rows_per_w

    table = jnp.concatenate([x_table, y_table, w_table, h_table], axis=0)

    coords_i = jnp.stack(
        [a.reshape(NB) for a in (xmin, ymin, xmax, ymax, width, height)],
        axis=-1).reshape(NB * NSEG)
    coords_bits = lax.bitcast_convert_type(coords_i, jnp.int32)
    fp = first_page.reshape(NB)
    lp = last_page.reshape(NB)

    mesh = plsc.VectorSubcoreMesh(core_axis_name="c", subcore_axis_name="s",
                                  num_cores=NC, num_subcores=NS)

    @functools.partial(
        pl.kernel,
        out_type=jax.ShapeDtypeStruct((NB * NSEG, SUB), jnp.float32),
        mesh=mesh,
        scratch_types=[
            pltpu.VMEM((KW,), jnp.int32),
            pltpu.VMEM((2, rows_per_w), jnp.float32),
            pltpu.VMEM((KPC, SUB), jnp.float32),
            pltpu.VMEM((2, SIZE), jnp.float32),
            pltpu.SemaphoreType.DMA,
        ],
    )
    def sc_kernel(table_h, coords_h, fp_h, lp_h, fpe_h, lpe_h, out_h,
                  idx_v, page_v, rows_v, pe_v, gsem):
        lane_iota = lax.broadcasted_iota(jnp.int32, (LANES,), 0)
        lane_bcast = [lane_iota * 0 + u for u in range(LANES)]
        wid = lax.axis_index("s") * NC + lax.axis_index("c")
        base_w = wid * rows_per_w
        pltpu.sync_copy(fpe_h, pe_v.at[0])
        pltpu.sync_copy(lpe_h, pe_v.at[1])
        pltpu.sync_copy(fp_h.at[pl.ds(base_w, rows_per_w)], page_v.at[0])
        pltpu.sync_copy(lp_h.at[pl.ds(base_w, rows_per_w)], page_v.at[1])
        pltpu.sync_copy(coords_h.at[pl.ds(base_w * NSEG, KW)], idx_v)

        def idx_body(t, _):
            sl = pl.ds(t * LANES, LANES)
            k = t * LANES + lane_iota
            s = k - 6 * ((k * 43691) >> 18)
            v = lax.bitcast_convert_type(idx_v[sl], jnp.float32)
            v = jnp.where(s == 5, v * 5.0, v)
            v = jnp.minimum(v * float(N_POS), float(N_POS - 1))
            off = jnp.where(s >= 4, s - 2, s & 1) * N_POS
            idx_v[sl] = v.astype(jnp.int32) + off
            return 0

        lax.fori_loop(0, KW // LANES, idx_body, 0)

        def chunk_body(c, carry):
            base = base_w + c * R
            pltpu.async_copy(table_h.at[idx_v.at[pl.ds(c * KPC, KPC)]],
                             rows_v, gsem).wait()
            for a in range(NSEG):
                fpe8 = [pe_v[0, pl.ds(a * SUB + j * LANES, LANES)]
                        for j in range(SUB // LANES)]
                lpe8 = [pe_v[1, pl.ds(a * SUB + j * LANES, LANES)]
                        for j in range(SUB // LANES)]

                def row_body(g, _, a=a, fpe8=fpe8, lpe8=lpe8):
                    gsl = pl.ds(c * R + g * LANES, LANES)
                    fp16 = page_v[0, gsl]
                    lp16 = page_v[1, gsl]
                    for u in range(LANES):
                        r = (g * LANES + u) * NSEG + a
                        fpi = jnp.take_along_axis(fp16, lane_bcast[u], axis=0)
                        lpi = jnp.take_along_axis(lp16, lane_bcast[u], axis=0)
                        for j in range(SUB // LANES):
                            sl = pl.ds(j * LANES, LANES)
                            rows_v[r, sl] = (rows_v[r, sl]
                                             + fpi * fpe8[j]
                                             + lpi * lpe8[j])
                    return 0

                lax.fori_loop(0, R // LANES, row_body, 0)
            pltpu.sync_copy(rows_v, out_h.at[pl.ds(base * NSEG, KPC)])
            return carry

        lax.fori_loop(0, n_chunks, chunk_body, 0)

    out = sc_kernel(table, coords_bits, fp, lp,
                    first_page_embedding, last_page_embedding)
    return out.reshape(B, L, SIZE)
